# baseline (speedup 1.0000x reference)
import math

import jax
import jax.numpy as jnp
from jax import lax
from jax.experimental import pallas as pl
from jax.experimental.pallas import tpu as pltpu
from jax.experimental.pallas import tpu_sc as plsc

VOCAB = 1000000
EMB = 64
SCALE = math.sqrt(EMB)
NC, NS = 2, 16
FPC = EMB // NC
BT = 256
TT = 200
TQ = TT // 8
NQ = 8
NSTG = 16
SLC = 62464
TAIL = VOCAB - NSTG * SLC


def _emb_inner(tokT_hbm, tabT_hbm, outT_hbm,
               idx_a, idx_b, vals0, vals1, tail_v, sp,
               sg0, sg1, ss0, ss1):
    c = lax.axis_index("c")
    s = lax.axis_index("s")
    vals = (vals0, vals1)
    sg = (sg0, sg1)
    ss = (ss0, ss1)
    b0 = s * BT
    e0 = c * FPC

    for k in range(NS):
        @pl.when(s == k)
        def _(k=k):
            pltpu.sync_copy(tokT_hbm.at[:, pl.ds(k * BT, 128)], idx_a)
            pltpu.sync_copy(tokT_hbm.at[:, pl.ds(k * BT + 128, 128)], idx_b)

    def fire_gather(q, vb):
        @pl.loop(0, TQ, unroll=5)
        def _t(tl):
            t = q * TQ + tl
            pltpu.async_copy(sp.at[idx_a.at[t]],
                             vals[vb].at[tl, pl.ds(0, 128)], sg[vb])
            pltpu.async_copy(sp.at[idx_b.at[t]],
                             vals[vb].at[tl, pl.ds(128, 128)], sg[vb])

    def wait_gather(q, vb):
        pltpu.make_async_copy(outT_hbm.at[pl.ds(0, TQ), 0, pl.ds(0, BT)],
                              vals[vb], sg[vb]).wait()

    def scale(vb):
        @pl.loop(0, TQ, unroll=5)
        def _r(i):
            for w in range(BT // 16):
                sl = pl.ds(w * 16, 16)
                vals[vb][i, sl] = vals[vb][i, sl] * SCALE

    def fire_store(e_local, q, vb):
        pltpu.async_copy(vals[vb],
                         outT_hbm.at[pl.ds(q * TQ, TQ), e0 + e_local,
                                     pl.ds(b0, BT)], ss[vb])

    def wait_store(e_local, q, vb):
        pltpu.make_async_copy(vals[vb],
                              outT_hbm.at[pl.ds(q * TQ, TQ), e0 + e_local,
                                          pl.ds(b0, BT)], ss[vb]).wait()

    @pl.loop(0, FPC)
    def _feature(e):
        plsc.subcore_barrier()

        @pl.when(s < NSTG)
        def _():
            off = s * SLC
            pltpu.sync_copy(tabT_hbm.at[e0 + e, pl.ds(off, SLC)],
                            sp.at[pl.ds(off, SLC)])

        @pl.when(s == 0)
        def _():
            pltpu.sync_copy(tabT_hbm.at[e0 + e, pl.ds(NSTG * SLC, TAIL - 64)],
                            sp.at[pl.ds(NSTG * SLC, TAIL - 64)])

        @pl.when(s == 1)
        def _():
            pltpu.sync_copy(tabT_hbm.at[pl.ds(e0 + e, 1), pl.ds(VOCAB - 64, 64)],
                            tail_v)
            pltpu.sync_copy(tail_v.at[0], sp.at[pl.ds(VOCAB - 64, 64)])

        plsc.subcore_barrier()

        for q in range(NQ):
            vb = q % 2

            @pl.when(e > 0)
            def _(q=q, vb=vb):
                wait_store(e - 1, (q - 2) % NQ, vb)

            @pl.when(e == 0)
            def _(q=q, vb=vb):
                @pl.when(q >= 2)
                def _():
                    wait_store(e, q - 2, vb)

            fire_gather(q, vb)
            if q > 0:
                wait_gather(q - 1, 1 - vb)
                scale(1 - vb)
                fire_store(e, q - 1, 1 - vb)

        wait_gather(NQ - 1, 1)
        scale(1)
        fire_store(e, NQ - 1, 1)

    wait_store(FPC - 1, NQ - 2, 0)
    wait_store(FPC - 1, NQ - 1, 1)


def kernel(tokens, table):
    b, t = tokens.shape
    tokT = jnp.transpose(tokens.astype(jnp.int32))
    tabT = jnp.transpose(table)
    grid_kernel = pl.kernel(
        _emb_inner,
        out_type=jax.ShapeDtypeStruct((t, EMB, b), jnp.float32),
        mesh=plsc.VectorSubcoreMesh(core_axis_name="c", subcore_axis_name="s"),
        compiler_params=pltpu.CompilerParams(use_tc_tiling_on_sc=True),
        scratch_types=[
            pltpu.VMEM((TT, 128), jnp.int32),
            pltpu.VMEM((TT, 128), jnp.int32),
            pltpu.VMEM((TQ, BT), jnp.float32),
            pltpu.VMEM((TQ, BT), jnp.float32),
            pltpu.VMEM((1, 64), jnp.float32),
            pltpu.VMEM_SHARED((VOCAB,), jnp.float32),
        ]
        + [pltpu.SemaphoreType.DMA for _ in range(4)],
    )
    outT = grid_kernel(tokT, tabT)
    return jnp.transpose(outT, (2, 0, 1))


# speedup vs baseline: 1.0182x; 1.0182x over previous
import math

import jax
import jax.numpy as jnp
from jax import lax
from jax.experimental import pallas as pl
from jax.experimental.pallas import tpu as pltpu
from jax.experimental.pallas import tpu_sc as plsc

VOCAB = 1000000
EMB = 64
SCALE = math.sqrt(EMB)
NC, NS = 2, 16
FPC = EMB // NC
BT = 256
TT = 200
TQ = TT // 8
NQ = 8
NSTG = 8
SLC = 124928
TAIL = VOCAB - NSTG * SLC


def _emb_inner(tokT_hbm, tabT_hbm, outT_hbm,
               idx_a, idx_b, vals0, vals1, tail_v, sp,
               sg0, sg1, ss0, ss1):
    c = lax.axis_index("c")
    s = lax.axis_index("s")
    vals = (vals0, vals1)
    sg = (sg0, sg1)
    ss = (ss0, ss1)
    b0 = s * BT
    e0 = c * FPC

    for k in range(NS):
        @pl.when(s == k)
        def _(k=k):
            pltpu.sync_copy(tokT_hbm.at[:, pl.ds(k * BT, 128)], idx_a)
            pltpu.sync_copy(tokT_hbm.at[:, pl.ds(k * BT + 128, 128)], idx_b)

    def fire_gather(q, vb):
        @pl.loop(0, TQ)
        def _t(tl):
            t = q * TQ + tl
            pltpu.async_copy(sp.at[idx_a.at[t]],
                             vals[vb].at[tl, pl.ds(0, 128)], sg[vb])
            pltpu.async_copy(sp.at[idx_b.at[t]],
                             vals[vb].at[tl, pl.ds(128, 128)], sg[vb])

    def wait_gather(q, vb):
        @pl.loop(0, TQ)
        def _t(tl):
            t = q * TQ + tl
            pltpu.make_async_copy(sp.at[idx_a.at[t]],
                                  vals[vb].at[tl, pl.ds(0, 128)], sg[vb]).wait()
            pltpu.make_async_copy(sp.at[idx_b.at[t]],
                                  vals[vb].at[tl, pl.ds(128, 128)], sg[vb]).wait()

    def scale(vb):
        @pl.loop(0, TQ, unroll=5)
        def _r(i):
            for w in range(BT // 16):
                sl = pl.ds(w * 16, 16)
                vals[vb][i, sl] = vals[vb][i, sl] * SCALE

    def fire_store(e_local, q, vb):
        pltpu.async_copy(vals[vb],
                         outT_hbm.at[pl.ds(q * TQ, TQ), e0 + e_local,
                                     pl.ds(b0, BT)], ss[vb])

    def wait_store(e_local, q, vb):
        pltpu.make_async_copy(vals[vb],
                              outT_hbm.at[pl.ds(q * TQ, TQ), e0 + e_local,
                                          pl.ds(b0, BT)], ss[vb]).wait()

    @pl.loop(0, FPC)
    def _feature(e):
        plsc.subcore_barrier()

        @pl.when(s < NSTG)
        def _():
            off = s * SLC
            pltpu.sync_copy(tabT_hbm.at[e0 + e, pl.ds(off, SLC)],
                            sp.at[pl.ds(off, SLC)])

        @pl.when(s == NSTG)
        def _():
            pltpu.sync_copy(tabT_hbm.at[e0 + e, pl.ds(NSTG * SLC, TAIL - 64)],
                            sp.at[pl.ds(NSTG * SLC, TAIL - 64)])

        @pl.when(s == NSTG + 1)
        def _():
            pltpu.sync_copy(tabT_hbm.at[pl.ds(e0 + e, 1), pl.ds(VOCAB - 64, 64)],
                            tail_v)
            pltpu.sync_copy(tail_v.at[0], sp.at[pl.ds(VOCAB - 64, 64)])

        plsc.subcore_barrier()

        for q in range(NQ):
            vb = q % 2

            @pl.when(e > 0)
            def _(q=q, vb=vb):
                wait_store(e - 1, (q - 2) % NQ, vb)

            @pl.when(e == 0)
            def _(q=q, vb=vb):
                @pl.when(q >= 2)
                def _():
                    wait_store(e, q - 2, vb)

            fire_gather(q, vb)
            if q > 0:
                wait_gather(q - 1, 1 - vb)
                scale(1 - vb)
                fire_store(e, q - 1, 1 - vb)

        wait_gather(NQ - 1, 1)
        scale(1)
        fire_store(e, NQ - 1, 1)

    wait_store(FPC - 1, NQ - 2, 0)
    wait_store(FPC - 1, NQ - 1, 1)


def kernel(tokens, table):
    b, t = tokens.shape
    tokT = jnp.transpose(tokens.astype(jnp.int32))
    tabT = jnp.transpose(table)
    grid_kernel = pl.kernel(
        _emb_inner,
        out_type=jax.ShapeDtypeStruct((t, EMB, b), jnp.float32),
        mesh=plsc.VectorSubcoreMesh(core_axis_name="c", subcore_axis_name="s"),
        compiler_params=pltpu.CompilerParams(use_tc_tiling_on_sc=True),
        scratch_types=[
            pltpu.VMEM((TT, 128), jnp.int32),
            pltpu.VMEM((TT, 128), jnp.int32),
            pltpu.VMEM((TQ, BT), jnp.float32),
            pltpu.VMEM((TQ, BT), jnp.float32),
            pltpu.VMEM((1, 64), jnp.float32),
            pltpu.VMEM_SHARED((VOCAB,), jnp.float32),
        ]
        + [pltpu.SemaphoreType.DMA for _ in range(4)],
    )
    outT = grid_kernel(tokT, tabT)
    return jnp.transpose(outT, (2, 0, 1))


# speedup vs baseline: 1.0183x; 1.0001x over previous
import math

import jax
import jax.numpy as jnp
from jax import lax
from jax.experimental import pallas as pl
from jax.experimental.pallas import tpu as pltpu
from jax.experimental.pallas import tpu_sc as plsc

VOCAB = 1000000
EMB = 64
SCALE = math.sqrt(EMB)
NC, NS = 2, 16
FPC = EMB // NC
BT = 256
TT = 200
TQ = TT // 8
NQ = 8
NSTG = 8
SLC = 124928
TAIL = VOCAB - NSTG * SLC


def _emb_inner(tokT_hbm, tabT_hbm, outT_hbm,
               idx_a, idx_b, vals0, vals1, tail_v, sp,
               sg0, sg1, ss0, ss1):
    c = lax.axis_index("c")
    s = lax.axis_index("s")
    vals = (vals0, vals1)
    sg = (sg0, sg1)
    ss = (ss0, ss1)
    b0 = s * BT
    e0 = c * FPC

    for k in range(NS):
        @pl.when(s == k)
        def _(k=k):
            pltpu.sync_copy(tokT_hbm.at[:, pl.ds(k * BT, 128)], idx_a)
            pltpu.sync_copy(tokT_hbm.at[:, pl.ds(k * BT + 128, 128)], idx_b)

    def fire_gather(q, vb):
        @pl.loop(0, TQ)
        def _t(tl):
            t = q * TQ + tl
            pltpu.async_copy(sp.at[idx_a.at[t]],
                             vals[vb].at[tl, pl.ds(0, 128)], sg[vb])
            pltpu.async_copy(sp.at[idx_b.at[t]],
                             vals[vb].at[tl, pl.ds(128, 128)], sg[vb])

    def wait_gather(q, vb):
        pltpu.make_async_copy(outT_hbm.at[pl.ds(0, TQ), 0, pl.ds(0, BT)],
                              vals[vb], sg[vb]).wait()

    def scale(vb):
        @pl.loop(0, TQ, unroll=5)
        def _r(i):
            for w in range(BT // 16):
                sl = pl.ds(w * 16, 16)
                vals[vb][i, sl] = vals[vb][i, sl] * SCALE

    def fire_store(e_local, q, vb):
        pltpu.async_copy(vals[vb],
                         outT_hbm.at[pl.ds(q * TQ, TQ), e0 + e_local,
                                     pl.ds(b0, BT)], ss[vb])

    def wait_store(e_local, q, vb):
        pltpu.make_async_copy(vals[vb],
                              outT_hbm.at[pl.ds(q * TQ, TQ), e0 + e_local,
                                          pl.ds(b0, BT)], ss[vb]).wait()

    @pl.loop(0, FPC)
    def _feature(e):
        plsc.subcore_barrier()

        @pl.when(s < NSTG)
        def _():
            off = s * SLC
            pltpu.sync_copy(tabT_hbm.at[e0 + e, pl.ds(off, SLC)],
                            sp.at[pl.ds(off, SLC)])

        @pl.when(s == NSTG)
        def _():
            pltpu.sync_copy(tabT_hbm.at[e0 + e, pl.ds(NSTG * SLC, TAIL - 64)],
                            sp.at[pl.ds(NSTG * SLC, TAIL - 64)])

        @pl.when(s == NSTG + 1)
        def _():
            pltpu.sync_copy(tabT_hbm.at[pl.ds(e0 + e, 1), pl.ds(VOCAB - 64, 64)],
                            tail_v)
            pltpu.sync_copy(tail_v.at[0], sp.at[pl.ds(VOCAB - 64, 64)])

        plsc.subcore_barrier()

        for q in range(NQ):
            vb = q % 2

            @pl.when(e > 0)
            def _(q=q, vb=vb):
                wait_store(e - 1, (q - 2) % NQ, vb)

            @pl.when(e == 0)
            def _(q=q, vb=vb):
                @pl.when(q >= 2)
                def _():
                    wait_store(e, q - 2, vb)

            fire_gather(q, vb)
            if q > 0:
                wait_gather(q - 1, 1 - vb)
                scale(1 - vb)
                fire_store(e, q - 1, 1 - vb)

        wait_gather(NQ - 1, 1)
        scale(1)
        fire_store(e, NQ - 1, 1)

    wait_store(FPC - 1, NQ - 2, 0)
    wait_store(FPC - 1, NQ - 1, 1)


def kernel(tokens, table):
    b, t = tokens.shape
    tokT = jnp.transpose(tokens.astype(jnp.int32))
    tabT = jnp.transpose(table)
    grid_kernel = pl.kernel(
        _emb_inner,
        out_type=jax.ShapeDtypeStruct((t, EMB, b), jnp.float32),
        mesh=plsc.VectorSubcoreMesh(core_axis_name="c", subcore_axis_name="s"),
        compiler_params=pltpu.CompilerParams(use_tc_tiling_on_sc=True),
        scratch_types=[
            pltpu.VMEM((TT, 128), jnp.int32),
            pltpu.VMEM((TT, 128), jnp.int32),
            pltpu.VMEM((TQ, BT), jnp.float32),
            pltpu.VMEM((TQ, BT), jnp.float32),
            pltpu.VMEM((1, 64), jnp.float32),
            pltpu.VMEM_SHARED((VOCAB,), jnp.float32),
        ]
        + [pltpu.SemaphoreType.DMA for _ in range(4)],
    )
    outT = grid_kernel(tokT, tabT)
    return jnp.transpose(outT, (2, 0, 1))
